# baseline (speedup 1.0000x reference)
import functools

import numpy as np

import jax
import jax.numpy as jnp
from jax import lax
from jax.experimental import pallas as pl
from jax.experimental.pallas import tpu as pltpu
from jax.experimental.pallas import tpu_sc as plsc

NN = 10000
EE = 160000
EI = 80000
HID = 128
NL = 4
CH = 128
NCH_E = EE // CH
NCH_I = EI // CH
STR_E = 40
STR_I = 24
PCH_E = 1280
PCH_I = 768
NWORK = 32
NTILES = 16
NPAD = 10240
NSTRIPE = NPAD // NTILES


_CPAD_Z = np.zeros((NN, HID), np.float32)
_ZEROSN = np.zeros((NPAD, HID), np.float32)
_SEL = np.repeat(np.eye(100, dtype=np.float32), 100, axis=1)
_PAD_E = np.zeros((PCH_E - NCH_E, CH), np.int32)
_PAD_I = np.zeros((PCH_I - NCH_I, CH), np.int32)


def _part(total, stride, wid):
    start = wid * stride
    cnt = jnp.clip(total - start, 0, stride)
    return start, cnt


NB = 3


def _gather_tbl(tbl, idx2d, out, nch_total, stride, idxbuf, dbuf,
                gsems, osems, wid):
    start, cnt = _part(nch_total, stride, wid)
    pltpu.sync_copy(idx2d.at[pl.ds(pl.multiple_of(start, 8), stride), :],
                    idxbuf.at[pl.ds(0, stride), :])

    def g_pair(j):
        slot = lax.rem(j, NB)
        return tbl.at[idxbuf.at[j]], dbuf.at[slot], gsems.at[slot]

    def o_pair(j):
        slot = lax.rem(j, NB)
        return (dbuf.at[slot], out.at[pl.ds((start + j) * CH, CH), :],
                osems.at[slot])

    def body(j, carry):
        @pl.when(j >= NB)
        def _():
            pltpu.make_async_copy(*o_pair(j - NB)).wait()

        pltpu.async_copy(*g_pair(j))

        @pl.when(j >= 1)
        def _():
            pltpu.make_async_copy(*g_pair(j - 1)).wait()
            pltpu.async_copy(*o_pair(j - 1))

        return carry

    lax.fori_loop(0, cnt, body, 0)

    @pl.when(cnt >= 1)
    def _():
        pltpu.make_async_copy(*g_pair(cnt - 1)).wait()
        pltpu.async_copy(*o_pair(cnt - 1))

    @pl.when(cnt >= 3)
    def _():
        pltpu.make_async_copy(*o_pair(cnt - 3)).wait()

    @pl.when(cnt >= 2)
    def _():
        pltpu.make_async_copy(*o_pair(cnt - 2)).wait()

    @pl.when(cnt >= 1)
    def _():
        pltpu.make_async_copy(*o_pair(cnt - 1)).wait()


def _sc_mesh():
    return plsc.VectorSubcoreMesh(core_axis_name="c", subcore_axis_name="s")


@functools.cache
def _make_gather2sum(rows, nch, stride, sign):
    @functools.partial(
        pl.kernel,
        out_type=jax.ShapeDtypeStruct((rows, HID), jnp.float32),
        scratch_types=[
            pltpu.VMEM((2, 40, CH), jnp.int32),
            pltpu.VMEM((2, CH, HID), jnp.float32),
            pltpu.VMEM((2, CH, HID), jnp.float32),
            pltpu.SemaphoreType.DMA((2,)),
            pltpu.SemaphoreType.DMA((2,)),
            pltpu.SemaphoreType.DMA((2,)),
        ],
        mesh=_sc_mesh(),
    )
    def _gather2sum(ta, tb, ia2, ib2, dep, out,
                    idxbuf, abuf, bbuf, asems, bsems, osems):
        del dep
        wid = lax.axis_index("s") * 2 + lax.axis_index("c")
        start, cnt = _part(nch, stride, wid)
        pltpu.sync_copy(ia2.at[pl.ds(pl.multiple_of(start, 8), stride), :],
                        idxbuf.at[0, pl.ds(0, stride), :])
        pltpu.sync_copy(ib2.at[pl.ds(pl.multiple_of(start, 8), stride), :],
                        idxbuf.at[1, pl.ds(0, stride), :])

        def a_pair(j):
            slot = lax.rem(j, 2)
            return ta.at[idxbuf.at[0, j]], abuf.at[slot], asems.at[slot]

        def b_pair(j):
            slot = lax.rem(j, 2)
            return tb.at[idxbuf.at[1, j]], bbuf.at[slot], bsems.at[slot]

        def o_pair(j):
            slot = lax.rem(j, 2)
            return (abuf.at[slot], out.at[pl.ds((start + j) * CH, CH), :],
                    osems.at[slot])

        def combine(j):
            slot = lax.rem(j, 2)

            def rowbody(i, carry):
                for g in range(HID // 16):
                    av = abuf[slot, i, pl.ds(g * 16, 16)]
                    bv = bbuf[slot, i, pl.ds(g * 16, 16)]
                    r = av + bv if sign > 0 else av - bv
                    abuf[slot, i, pl.ds(g * 16, 16)] = r
                return carry

            lax.fori_loop(0, CH, rowbody, 0)

        def body(j, carry):
            @pl.when(j >= 2)
            def _():
                pltpu.make_async_copy(*o_pair(j - 2)).wait()

            pltpu.async_copy(*a_pair(j))
            pltpu.async_copy(*b_pair(j))

            @pl.when(j >= 1)
            def _():
                pltpu.make_async_copy(*a_pair(j - 1)).wait()
                pltpu.make_async_copy(*b_pair(j - 1)).wait()
                combine(j - 1)
                pltpu.async_copy(*o_pair(j - 1))

            return carry

        lax.fori_loop(0, cnt, body, 0)

        @pl.when(cnt >= 1)
        def _():
            pltpu.make_async_copy(*a_pair(cnt - 1)).wait()
            pltpu.make_async_copy(*b_pair(cnt - 1)).wait()
            combine(cnt - 1)
            pltpu.async_copy(*o_pair(cnt - 1))

        @pl.when(cnt >= 2)
        def _():
            pltpu.make_async_copy(*o_pair(cnt - 2)).wait()

        @pl.when(cnt >= 1)
        def _():
            pltpu.make_async_copy(*o_pair(cnt - 1)).wait()

    return _gather2sum


@functools.cache
def _make_scatter1(nch, stride):
    @functools.partial(
        pl.kernel,
        out_type=jax.ShapeDtypeStruct((2 * NPAD, HID), jnp.float32),
        scratch_types=[
            pltpu.VMEM((40, CH), jnp.int32),
            pltpu.VMEM((2, CH, HID), jnp.float32),
            pltpu.VMEM_SHARED((NPAD, HID), jnp.float32),
            pltpu.SemaphoreType.DMA((2,)),
            pltpu.SemaphoreType.DMA((2,)),
        ],
        mesh=_sc_mesh(),
    )
    def _scatter1(msgs, idx2d, zerosn, dep, aggp, ibuf, mbuf, acc,
                  lsems, ssems):
        del dep
        c = lax.axis_index("c")
        s = lax.axis_index("s")
        rstart = s * NSTRIPE
        pltpu.sync_copy(zerosn.at[pl.ds(rstart, NSTRIPE), :],
                        acc.at[pl.ds(rstart, NSTRIPE), :])
        plsc.subcore_barrier()

        wid = s * 2 + c
        ts, tcnt = _part(nch, stride, wid)
        pltpu.sync_copy(idx2d.at[pl.ds(pl.multiple_of(ts, 8), stride), :],
                        ibuf.at[pl.ds(0, stride), :])

        def l_pair(j):
            slot = lax.rem(j, 2)
            return (msgs.at[pl.ds((ts + j) * CH, CH), :], mbuf.at[slot],
                    lsems.at[slot])

        def s_pair(j):
            slot = lax.rem(j, 2)
            return mbuf.at[slot], acc.at[ibuf.at[j]], ssems.at[slot]

        @pl.when(tcnt >= 1)
        def _():
            pltpu.async_copy(*l_pair(0))

        def body(j, carry):
            @pl.when(j >= 2)
            def _():
                pltpu.make_async_copy(*s_pair(j - 2)).wait()

            @pl.when(j + 1 < tcnt)
            def _():
                pltpu.async_copy(*l_pair(j + 1))

            pltpu.make_async_copy(*l_pair(j)).wait()
            pltpu.async_copy(*s_pair(j), add=True)
            return carry

        lax.fori_loop(0, tcnt, body, 0)

        @pl.when(tcnt >= 2)
        def _():
            pltpu.make_async_copy(*s_pair(tcnt - 2)).wait()

        @pl.when(tcnt >= 1)
        def _():
            pltpu.make_async_copy(*s_pair(tcnt - 1)).wait()

        plsc.subcore_barrier()
        pltpu.sync_copy(acc.at[pl.ds(rstart, NSTRIPE), :],
                        aggp.at[pl.ds(c * NPAD + rstart, NSTRIPE), :])

    return _scatter1


def _full(shape):
    return pl.BlockSpec(shape, lambda i: tuple(0 for _ in shape))


def _rows(bs, w):
    return pl.BlockSpec((bs, w), lambda i: (i, 0))


BI = 1000


def _init_call(h, int_h, emb_W, emb_b, e1a, e1b, i1a, i1b):
    def f(h_ref, hi_ref, w_ref, b_ref, ea_ref, eb_ref, ia_ref, ib_ref,
          x_ref, xi_ref, xs_ref, xc_ref, xsi_ref, xci_ref):
        dot = lambda a, b: jnp.dot(a, b, preferred_element_type=jnp.float32,
                                   precision=jax.lax.Precision.HIGHEST)
        x = dot(h_ref[...], w_ref[...]) + b_ref[...]
        xi = dot(hi_ref[...], w_ref[...]) + b_ref[...]
        x_ref[...] = x
        xi_ref[...] = xi
        xs_ref[...] = dot(x, ea_ref[...])
        xc_ref[...] = dot(x, eb_ref[...])
        xsi_ref[...] = dot(x, ia_ref[...])
        xci_ref[...] = dot(xi, ib_ref[...])

    sds = jax.ShapeDtypeStruct((NN, HID), jnp.float32)
    return pl.pallas_call(
        f,
        grid=(NN // BI,),
        in_specs=[_rows(BI, HID), _rows(BI, HID), _full((HID, HID)),
                  _full((1, HID)), _full((HID, HID)), _full((HID, HID)),
                  _full((HID, HID)), _full((HID, HID))],
        out_specs=[_rows(BI, HID)] * 6,
        out_shape=[sds] * 6,
    )(h, int_h, emb_W, emb_b, e1a, e1b, i1a, i1b)


BE = 2000


def _rea_prep(gcd, edge_attr):
    def f(gcd_ref, ea_ref, out_ref):
        d = gcd_ref[...]
        r = jnp.sum(d * d, axis=1, keepdims=True)
        out_ref[...] = jnp.concatenate(
            [r, ea_ref[...], jnp.zeros((BE, 15), jnp.float32)], axis=1)

    return pl.pallas_call(
        f,
        grid=(EE // BE,),
        in_specs=[_rows(BE, HID), _rows(BE, 16)],
        out_specs=_rows(BE, 32),
        out_shape=jax.ShapeDtypeStruct((EE, 32), jnp.float32),
    )(gcd, edge_attr)


def _edge_mlp(pre, rea, Wr, b1, W2, b2):
    def f(a_ref, r_ref, wr_ref, b1_ref, w2_ref, b2_ref, m_ref):
        dot = lambda a, b: jnp.dot(a, b, preferred_element_type=jnp.float32,
                                   precision=jax.lax.Precision.HIGHEST)
        u = a_ref[...] + dot(r_ref[...], wr_ref[...]) + b1_ref[...]
        u = jnp.maximum(u, 0.0)
        m_ref[...] = jnp.maximum(dot(u, w2_ref[...]) + b2_ref[...], 0.0)

    return pl.pallas_call(
        f,
        grid=(EE // BE,),
        in_specs=[_rows(BE, HID), _rows(BE, 32),
                  _full((32, HID)), _full((1, HID)), _full((HID, HID)),
                  _full((1, HID))],
        out_specs=_rows(BE, HID),
        out_shape=jax.ShapeDtypeStruct((EE, HID), jnp.float32),
    )(pre, rea, Wr, b1, W2, b2)


def _int_mlp(pre, b1, W2, b2):
    def f(a_ref, b1_ref, w2_ref, b2_ref, m_ref):
        dot = lambda a, b: jnp.dot(a, b, preferred_element_type=jnp.float32,
                                   precision=jax.lax.Precision.HIGHEST)
        u = jnp.maximum(a_ref[...] + b1_ref[...], 0.0)
        m_ref[...] = jnp.maximum(dot(u, w2_ref[...]) + b2_ref[...], 0.0)

    return pl.pallas_call(
        f,
        grid=(EI // BE,),
        in_specs=[_rows(BE, HID), _full((1, HID)),
                  _full((HID, HID)), _full((1, HID))],
        out_specs=_rows(BE, HID),
        out_shape=jax.ShapeDtypeStruct((EI, HID), jnp.float32),
    )(pre, b1, W2, b2)


def _node_mlp(x, p0, p1, p2, p3, node_attr, xi, A1, A2, A3, b1, W2, b2,
              ne1a, ne1b, ni1a, ni1b, last):
    def f(x_ref, p0_ref, p1_ref, p2_ref, p3_ref, na_ref, xi_ref,
          a1_ref, a2_ref, a3_ref, b1_ref, w2_ref, b2_ref,
          ea_ref, eb_ref, ia_ref, ib_ref, *out_refs):
        dot = lambda a, b: jnp.dot(a, b, preferred_element_type=jnp.float32,
                                   precision=jax.lax.Precision.HIGHEST)
        agg = (p0_ref[...] + p1_ref[...]) + (p2_ref[...] + p3_ref[...])
        z = dot(x_ref[...], a1_ref[...]) + dot(agg, a2_ref[...]) \
            + dot(na_ref[...], a3_ref[...]) + b1_ref[...]
        z = jnp.maximum(z, 0.0)
        xn = dot(z, w2_ref[...]) + b2_ref[...]
        out_refs[0][...] = xn
        if not last:
            out_refs[1][...] = dot(xn, ea_ref[...])
            out_refs[2][...] = dot(xn, eb_ref[...])
            out_refs[3][...] = dot(xn, ia_ref[...])
            out_refs[4][...] = dot(xi_ref[...], ib_ref[...])

    n_out = 1 if last else 5
    sds = jax.ShapeDtypeStruct((NN, HID), jnp.float32)
    res = pl.pallas_call(
        f,
        grid=(NN // BI,),
        in_specs=[_rows(BI, HID)] * 7
        + [_full((HID, HID))] * 3
        + [_full((1, HID)), _full((HID, HID)), _full((1, HID))]
        + [_full((HID, HID))] * 4,
        out_specs=[_rows(BI, HID)] * n_out,
        out_shape=[sds] * n_out,
    )(x, p0, p1, p2, p3, node_attr, xi, A1, A2, A3, b1, W2, b2,
      ne1a, ne1b, ni1a, ni1b)
    return res if not last else (res[0], None, None, None, None)


def _decoder(x, node_mask, sel, dW1, db1, dW2, db2):
    def f(x_ref, m_ref, sel_ref, w1_ref, b1_ref, w2_ref, b2_ref, out_ref):
        dot = lambda a, b: jnp.dot(a, b, preferred_element_type=jnp.float32,
                                   precision=jax.lax.Precision.HIGHEST)
        xm = x_ref[...] * m_ref[...]
        g = dot(sel_ref[...], xm)
        t = jnp.maximum(dot(g, w1_ref[...]) + b1_ref[...], 0.0)
        out_ref[...] = dot(t, w2_ref[...]) + b2_ref[...]

    return pl.pallas_call(
        f,
        grid=(1,),
        in_specs=[_full((NN, HID)), _full((NN, 1)), _full((100, NN)),
                  _full((HID, 2 * HID)), _full((1, 2 * HID)),
                  _full((2 * HID, 1)), _full((1, 1))],
        out_specs=_full((100, 1)),
        out_shape=jax.ShapeDtypeStruct((100, 1), jnp.float32),
    )(x, node_mask, sel, dW1, db1, dW2, db2)


def kernel(h, edges, edge_attr, node_attr, coord, n_nodes_h, node_mask,
           int_h, int_edges, emb_W, emb_b, edge_W1, edge_b1, edge_W2,
           edge_b2, int_W1, int_b1, int_W2, int_b2, node_W1, node_b1,
           node_W2, node_b2, dec_W1, dec_b1, dec_W2, dec_b2):
    f32 = jnp.float32
    row2 = jnp.concatenate([edges[0].reshape(NCH_E, CH), _PAD_E])
    col2 = jnp.concatenate([edges[1].reshape(NCH_E, CH), _PAD_E])
    irow2 = jnp.concatenate([int_edges[0].reshape(NCH_I, CH), _PAD_I])
    icol2 = jnp.concatenate([int_edges[1].reshape(NCH_I, CH), _PAD_I])
    cpad = jnp.asarray(_CPAD_Z).at[:, :3].set(coord)
    zerosn = _ZEROSN
    sel = _SEL

    e1a = [edge_W1[l, :HID, :] for l in range(NL)]
    e1b = [edge_W1[l, HID:2 * HID, :] for l in range(NL)]
    wr = [jnp.concatenate([edge_W1[l, 2 * HID:2 * HID + 17, :],
                           jnp.zeros((15, HID), f32)], axis=0)
          for l in range(NL)]
    eb1 = [edge_b1[l].reshape(1, HID) for l in range(NL)]
    eb2 = [edge_b2[l].reshape(1, HID) for l in range(NL)]
    i1a = [int_W1[l, :HID, :] for l in range(NL)]
    i1b = [int_W1[l, HID:, :] for l in range(NL)]
    ib1 = [int_b1[l].reshape(1, HID) for l in range(NL)]
    ib2 = [int_b2[l].reshape(1, HID) for l in range(NL)]
    nA1 = [node_W1[l, :HID, :] for l in range(NL)]
    nA2 = [node_W1[l, HID:2 * HID, :] for l in range(NL)]
    nA3 = [node_W1[l, 2 * HID:, :] for l in range(NL)]
    nb1 = [node_b1[l].reshape(1, HID) for l in range(NL)]
    nb2 = [node_b2[l].reshape(1, HID) for l in range(NL)]
    emb_b2 = emb_b.reshape(1, HID)
    db1 = dec_b1.reshape(1, 2 * HID)
    db2 = dec_b2.reshape(1, 1)

    gather_e = _make_gather2sum(EE, NCH_E, STR_E, 1)
    gather_i = _make_gather2sum(EI, NCH_I, STR_I, 1)
    gather_d = _make_gather2sum(EE, NCH_E, STR_E, -1)
    scatter_e = _make_scatter1(NCH_E, STR_E)
    scatter_i = _make_scatter1(NCH_I, STR_I)

    x, xi, xs, xc, xsi, xci = _init_call(
        h, int_h, emb_W, emb_b2, e1a[0], e1b[0], i1a[0], i1b[0])
    tok = lambda a: lax.slice(a, (0, 0), (8, HID))
    gcd = gather_d(cpad, cpad, row2, col2, tok(cpad))
    rea = _rea_prep(gcd, edge_attr)
    sc_tok = tok(gcd)

    for l in range(NL):
        pre_i = gather_i(xsi, xci, irow2, icol2, sc_tok)
        pre_e = gather_e(xs, xc, row2, col2, tok(pre_i))
        im = _int_mlp(pre_i, ib1[l], int_W2[l], ib2[l])
        m = _edge_mlp(pre_e, rea, wr[l], eb1[l], edge_W2[l], eb2[l])
        aggi = scatter_i(im, irow2, zerosn, tok(pre_e))
        agge = scatter_e(m, row2, zerosn, tok(aggi))
        sc_tok = tok(agge)
        ln = min(l + 1, NL - 1)
        x, xs, xc, xsi, xci = _node_mlp(
            x, agge[:NN], agge[NPAD:NPAD + NN],
            aggi[:NN], aggi[NPAD:NPAD + NN], node_attr, xi,
            nA1[l], nA2[l], nA3[l], nb1[l], node_W2[l], nb2[l],
            e1a[ln], e1b[ln], i1a[ln], i1b[ln], l == NL - 1)

    pred = _decoder(x, node_mask, sel, dec_W1, db1, dec_W2, db2)
    return pred.reshape(100)


# speedup vs baseline: 1.3753x; 1.3753x over previous
import functools

import numpy as np

import jax
import jax.numpy as jnp
from jax import lax
from jax.experimental import pallas as pl
from jax.experimental.pallas import tpu as pltpu
from jax.experimental.pallas import tpu_sc as plsc

NN = 10000
EE = 160000
EI = 80000
HID = 128
NL = 4
CH = 128
NCH_E = EE // CH
NCH_I = EI // CH
STR_E = 40
STR_I = 24
PCH_E = 1280
PCH_I = 768
NWORK = 32
NTILES = 16
NPAD = 10240
NSTRIPE = NPAD // NTILES


_CPAD_Z = np.zeros((NN, HID), np.float32)
_ZEROSN = np.zeros((NPAD, HID), np.float32)
_SEL = np.repeat(np.eye(100, dtype=np.float32), 100, axis=1)
_PAD_E = np.zeros((PCH_E - NCH_E, CH), np.int32)
_PAD_I = np.zeros((PCH_I - NCH_I, CH), np.int32)


def _part(total, stride, wid):
    start = wid * stride
    cnt = jnp.clip(total - start, 0, stride)
    return start, cnt


NB = 3


def _gather_tbl(tbl, idx2d, out, nch_total, stride, idxbuf, dbuf,
                gsems, osems, wid):
    start, cnt = _part(nch_total, stride, wid)
    pltpu.sync_copy(idx2d.at[pl.ds(pl.multiple_of(start, 8), stride), :],
                    idxbuf.at[pl.ds(0, stride), :])

    def g_pair(j):
        slot = lax.rem(j, NB)
        return tbl.at[idxbuf.at[j]], dbuf.at[slot], gsems.at[slot]

    def o_pair(j):
        slot = lax.rem(j, NB)
        return (dbuf.at[slot], out.at[pl.ds((start + j) * CH, CH), :],
                osems.at[slot])

    def body(j, carry):
        @pl.when(j >= NB)
        def _():
            pltpu.make_async_copy(*o_pair(j - NB)).wait()

        pltpu.async_copy(*g_pair(j))

        @pl.when(j >= 1)
        def _():
            pltpu.make_async_copy(*g_pair(j - 1)).wait()
            pltpu.async_copy(*o_pair(j - 1))

        return carry

    lax.fori_loop(0, cnt, body, 0)

    @pl.when(cnt >= 1)
    def _():
        pltpu.make_async_copy(*g_pair(cnt - 1)).wait()
        pltpu.async_copy(*o_pair(cnt - 1))

    @pl.when(cnt >= 3)
    def _():
        pltpu.make_async_copy(*o_pair(cnt - 3)).wait()

    @pl.when(cnt >= 2)
    def _():
        pltpu.make_async_copy(*o_pair(cnt - 2)).wait()

    @pl.when(cnt >= 1)
    def _():
        pltpu.make_async_copy(*o_pair(cnt - 1)).wait()


def _sc_mesh():
    return plsc.VectorSubcoreMesh(core_axis_name="c", subcore_axis_name="s")


@functools.cache
def _make_gather2(rows, nch, stride):
    @functools.partial(
        pl.kernel,
        out_type=[
            jax.ShapeDtypeStruct((rows, HID), jnp.float32),
            jax.ShapeDtypeStruct((rows, HID), jnp.float32),
        ],
        scratch_types=[
            pltpu.VMEM((40, CH), jnp.int32),
            pltpu.VMEM((NB, CH, HID), jnp.float32),
            pltpu.SemaphoreType.DMA((NB,)),
            pltpu.SemaphoreType.DMA((NB,)),
        ],
        mesh=_sc_mesh(),
    )
    def _gather2(ta, tb, ia2, ib2, dep, oa, ob, idxbuf, dbuf, gsems, osems):
        del dep
        wid = lax.axis_index("s") * 2 + lax.axis_index("c")
        _gather_tbl(ta, ia2, oa, nch, stride, idxbuf, dbuf, gsems, osems, wid)
        _gather_tbl(tb, ib2, ob, nch, stride, idxbuf, dbuf, gsems, osems, wid)

    return _gather2


@functools.cache
def _make_scatter1(nch, stride):
    @functools.partial(
        pl.kernel,
        out_type=jax.ShapeDtypeStruct((2 * NPAD, HID), jnp.float32),
        scratch_types=[
            pltpu.VMEM((40, CH), jnp.int32),
            pltpu.VMEM((2, CH, HID), jnp.float32),
            pltpu.VMEM_SHARED((NPAD, HID), jnp.float32),
            pltpu.SemaphoreType.DMA((2,)),
            pltpu.SemaphoreType.DMA((2,)),
        ],
        mesh=_sc_mesh(),
    )
    def _scatter1(msgs, idx2d, zerosn, dep, aggp, ibuf, mbuf, acc,
                  lsems, ssems):
        del dep
        c = lax.axis_index("c")
        s = lax.axis_index("s")
        rstart = s * NSTRIPE
        pltpu.sync_copy(zerosn.at[pl.ds(rstart, NSTRIPE), :],
                        acc.at[pl.ds(rstart, NSTRIPE), :])
        plsc.subcore_barrier()

        wid = s * 2 + c
        ts, tcnt = _part(nch, stride, wid)
        pltpu.sync_copy(idx2d.at[pl.ds(pl.multiple_of(ts, 8), stride), :],
                        ibuf.at[pl.ds(0, stride), :])

        def l_pair(j):
            slot = lax.rem(j, 2)
            return (msgs.at[pl.ds((ts + j) * CH, CH), :], mbuf.at[slot],
                    lsems.at[slot])

        def s_pair(j):
            slot = lax.rem(j, 2)
            return mbuf.at[slot], acc.at[ibuf.at[j]], ssems.at[slot]

        @pl.when(tcnt >= 1)
        def _():
            pltpu.async_copy(*l_pair(0))

        def body(j, carry):
            @pl.when(j >= 2)
            def _():
                pltpu.make_async_copy(*s_pair(j - 2)).wait()

            @pl.when(j + 1 < tcnt)
            def _():
                pltpu.async_copy(*l_pair(j + 1))

            pltpu.make_async_copy(*l_pair(j)).wait()
            pltpu.async_copy(*s_pair(j), add=True)
            return carry

        lax.fori_loop(0, tcnt, body, 0)

        @pl.when(tcnt >= 2)
        def _():
            pltpu.make_async_copy(*s_pair(tcnt - 2)).wait()

        @pl.when(tcnt >= 1)
        def _():
            pltpu.make_async_copy(*s_pair(tcnt - 1)).wait()

        plsc.subcore_barrier()
        pltpu.sync_copy(acc.at[pl.ds(rstart, NSTRIPE), :],
                        aggp.at[pl.ds(c * NPAD + rstart, NSTRIPE), :])

    return _scatter1


def _full(shape):
    return pl.BlockSpec(shape, lambda i: tuple(0 for _ in shape))


def _rows(bs, w):
    return pl.BlockSpec((bs, w), lambda i: (i, 0))


BI = 2000


def _init_call(h, int_h, emb_W, emb_b, e1a, e1b, i1a, i1b):
    def f(h_ref, hi_ref, w_ref, b_ref, ea_ref, eb_ref, ia_ref, ib_ref,
          x_ref, xi_ref, xs_ref, xc_ref, xsi_ref, xci_ref):
        dot = lambda a, b: jnp.dot(a, b, preferred_element_type=jnp.float32,
                                   precision=jax.lax.Precision.HIGHEST)
        x = dot(h_ref[...], w_ref[...]) + b_ref[...]
        xi = dot(hi_ref[...], w_ref[...]) + b_ref[...]
        x_ref[...] = x
        xi_ref[...] = xi
        xs_ref[...] = dot(x, ea_ref[...])
        xc_ref[...] = dot(x, eb_ref[...])
        xsi_ref[...] = dot(x, ia_ref[...])
        xci_ref[...] = dot(xi, ib_ref[...])

    sds = jax.ShapeDtypeStruct((NN, HID), jnp.float32)
    return pl.pallas_call(
        f,
        grid=(NN // BI,),
        in_specs=[_rows(BI, HID), _rows(BI, HID), _full((HID, HID)),
                  _full((1, HID)), _full((HID, HID)), _full((HID, HID)),
                  _full((HID, HID)), _full((HID, HID))],
        out_specs=[_rows(BI, HID)] * 6,
        out_shape=[sds] * 6,
    )(h, int_h, emb_W, emb_b, e1a, e1b, i1a, i1b)


BE = 4000


def _rea_prep(gcr, gcc, edge_attr):
    def f(gcr_ref, gcc_ref, ea_ref, out_ref):
        d = gcr_ref[...] - gcc_ref[...]
        r = jnp.sum(d * d, axis=1, keepdims=True)
        out_ref[...] = jnp.concatenate(
            [r, ea_ref[...], jnp.zeros((BE, 15), jnp.float32)], axis=1)

    return pl.pallas_call(
        f,
        grid=(EE // BE,),
        in_specs=[_rows(BE, HID), _rows(BE, HID), _rows(BE, 16)],
        out_specs=_rows(BE, 32),
        out_shape=jax.ShapeDtypeStruct((EE, 32), jnp.float32),
    )(gcr, gcc, edge_attr)


def _edge_mlp(gxs, gxc, rea, Wr, b1, W2, b2):
    def f(a_ref, c_ref, r_ref, wr_ref, b1_ref, w2_ref, b2_ref, m_ref):
        dot = lambda a, b: jnp.dot(a, b, preferred_element_type=jnp.float32,
                                   precision=jax.lax.Precision.HIGHEST)
        u = a_ref[...] + c_ref[...] + dot(r_ref[...], wr_ref[...]) + b1_ref[...]
        u = jnp.maximum(u, 0.0)
        m_ref[...] = jnp.maximum(dot(u, w2_ref[...]) + b2_ref[...], 0.0)

    return pl.pallas_call(
        f,
        grid=(EE // BE,),
        in_specs=[_rows(BE, HID), _rows(BE, HID), _rows(BE, 32),
                  _full((32, HID)), _full((1, HID)), _full((HID, HID)),
                  _full((1, HID))],
        out_specs=_rows(BE, HID),
        out_shape=jax.ShapeDtypeStruct((EE, HID), jnp.float32),
    )(gxs, gxc, rea, Wr, b1, W2, b2)


def _int_mlp(gxi, gxic, b1, W2, b2):
    def f(a_ref, c_ref, b1_ref, w2_ref, b2_ref, m_ref):
        dot = lambda a, b: jnp.dot(a, b, preferred_element_type=jnp.float32,
                                   precision=jax.lax.Precision.HIGHEST)
        u = jnp.maximum(a_ref[...] + c_ref[...] + b1_ref[...], 0.0)
        m_ref[...] = jnp.maximum(dot(u, w2_ref[...]) + b2_ref[...], 0.0)

    return pl.pallas_call(
        f,
        grid=(EI // BE,),
        in_specs=[_rows(BE, HID), _rows(BE, HID), _full((1, HID)),
                  _full((HID, HID)), _full((1, HID))],
        out_specs=_rows(BE, HID),
        out_shape=jax.ShapeDtypeStruct((EI, HID), jnp.float32),
    )(gxi, gxic, b1, W2, b2)


def _node_mlp(x, p0, p1, p2, p3, node_attr, xi, A1, A2, A3, b1, W2, b2,
              ne1a, ne1b, ni1a, ni1b, last):
    def f(x_ref, p0_ref, p1_ref, p2_ref, p3_ref, na_ref, xi_ref,
          a1_ref, a2_ref, a3_ref, b1_ref, w2_ref, b2_ref,
          ea_ref, eb_ref, ia_ref, ib_ref, *out_refs):
        dot = lambda a, b: jnp.dot(a, b, preferred_element_type=jnp.float32,
                                   precision=jax.lax.Precision.HIGHEST)
        agg = (p0_ref[...] + p1_ref[...]) + (p2_ref[...] + p3_ref[...])
        z = dot(x_ref[...], a1_ref[...]) + dot(agg, a2_ref[...]) \
            + dot(na_ref[...], a3_ref[...]) + b1_ref[...]
        z = jnp.maximum(z, 0.0)
        xn = dot(z, w2_ref[...]) + b2_ref[...]
        out_refs[0][...] = xn
        if not last:
            out_refs[1][...] = dot(xn, ea_ref[...])
            out_refs[2][...] = dot(xn, eb_ref[...])
            out_refs[3][...] = dot(xn, ia_ref[...])
            out_refs[4][...] = dot(xi_ref[...], ib_ref[...])

    n_out = 1 if last else 5
    sds = jax.ShapeDtypeStruct((NN, HID), jnp.float32)
    res = pl.pallas_call(
        f,
        grid=(NN // BI,),
        in_specs=[_rows(BI, HID)] * 7
        + [_full((HID, HID))] * 3
        + [_full((1, HID)), _full((HID, HID)), _full((1, HID))]
        + [_full((HID, HID))] * 4,
        out_specs=[_rows(BI, HID)] * n_out,
        out_shape=[sds] * n_out,
    )(x, p0, p1, p2, p3, node_attr, xi, A1, A2, A3, b1, W2, b2,
      ne1a, ne1b, ni1a, ni1b)
    return res if not last else (res[0], None, None, None, None)


def _decoder(x, node_mask, sel, dW1, db1, dW2, db2):
    def f(x_ref, m_ref, sel_ref, w1_ref, b1_ref, w2_ref, b2_ref, out_ref):
        dot = lambda a, b: jnp.dot(a, b, preferred_element_type=jnp.float32,
                                   precision=jax.lax.Precision.HIGHEST)
        xm = x_ref[...] * m_ref[...]
        g = dot(sel_ref[...], xm)
        t = jnp.maximum(dot(g, w1_ref[...]) + b1_ref[...], 0.0)
        out_ref[...] = dot(t, w2_ref[...]) + b2_ref[...]

    return pl.pallas_call(
        f,
        grid=(1,),
        in_specs=[_full((NN, HID)), _full((NN, 1)), _full((100, NN)),
                  _full((HID, 2 * HID)), _full((1, 2 * HID)),
                  _full((2 * HID, 1)), _full((1, 1))],
        out_specs=_full((100, 1)),
        out_shape=jax.ShapeDtypeStruct((100, 1), jnp.float32),
    )(x, node_mask, sel, dW1, db1, dW2, db2)


def kernel(h, edges, edge_attr, node_attr, coord, n_nodes_h, node_mask,
           int_h, int_edges, emb_W, emb_b, edge_W1, edge_b1, edge_W2,
           edge_b2, int_W1, int_b1, int_W2, int_b2, node_W1, node_b1,
           node_W2, node_b2, dec_W1, dec_b1, dec_W2, dec_b2):
    f32 = jnp.float32
    row2 = jnp.concatenate([edges[0].reshape(NCH_E, CH), _PAD_E])
    col2 = jnp.concatenate([edges[1].reshape(NCH_E, CH), _PAD_E])
    irow2 = jnp.concatenate([int_edges[0].reshape(NCH_I, CH), _PAD_I])
    icol2 = jnp.concatenate([int_edges[1].reshape(NCH_I, CH), _PAD_I])
    cpad = jnp.asarray(_CPAD_Z).at[:, :3].set(coord)
    zerosn = _ZEROSN
    sel = _SEL

    e1a = [edge_W1[l, :HID, :] for l in range(NL)]
    e1b = [edge_W1[l, HID:2 * HID, :] for l in range(NL)]
    wr = [jnp.concatenate([edge_W1[l, 2 * HID:2 * HID + 17, :],
                           jnp.zeros((15, HID), f32)], axis=0)
          for l in range(NL)]
    eb1 = [edge_b1[l].reshape(1, HID) for l in range(NL)]
    eb2 = [edge_b2[l].reshape(1, HID) for l in range(NL)]
    i1a = [int_W1[l, :HID, :] for l in range(NL)]
    i1b = [int_W1[l, HID:, :] for l in range(NL)]
    ib1 = [int_b1[l].reshape(1, HID) for l in range(NL)]
    ib2 = [int_b2[l].reshape(1, HID) for l in range(NL)]
    nA1 = [node_W1[l, :HID, :] for l in range(NL)]
    nA2 = [node_W1[l, HID:2 * HID, :] for l in range(NL)]
    nA3 = [node_W1[l, 2 * HID:, :] for l in range(NL)]
    nb1 = [node_b1[l].reshape(1, HID) for l in range(NL)]
    nb2 = [node_b2[l].reshape(1, HID) for l in range(NL)]
    emb_b2 = emb_b.reshape(1, HID)
    db1 = dec_b1.reshape(1, 2 * HID)
    db2 = dec_b2.reshape(1, 1)

    gather_e = _make_gather2(EE, NCH_E, STR_E)
    gather_i = _make_gather2(EI, NCH_I, STR_I)
    scatter_e = _make_scatter1(NCH_E, STR_E)
    scatter_i = _make_scatter1(NCH_I, STR_I)

    x, xi, xs, xc, xsi, xci = _init_call(
        h, int_h, emb_W, emb_b2, e1a[0], e1b[0], i1a[0], i1b[0])
    tok = lambda a: lax.slice(a, (0, 0), (8, HID))
    gcr, gcc = gather_e(cpad, cpad, row2, col2, tok(cpad))
    rea = _rea_prep(gcr, gcc, edge_attr)
    sc_tok = tok(gcr)

    for l in range(NL):
        gxi, gxic = gather_i(xsi, xci, irow2, icol2, sc_tok)
        gxs, gxc = gather_e(xs, xc, row2, col2, tok(gxi))
        im = _int_mlp(gxi, gxic, ib1[l], int_W2[l], ib2[l])
        m = _edge_mlp(gxs, gxc, rea, wr[l], eb1[l], edge_W2[l], eb2[l])
        aggi = scatter_i(im, irow2, zerosn, tok(gxs))
        agge = scatter_e(m, row2, zerosn, tok(aggi))
        sc_tok = tok(agge)
        ln = min(l + 1, NL - 1)
        x, xs, xc, xsi, xci = _node_mlp(
            x, agge[:NN], agge[NPAD:NPAD + NN],
            aggi[:NN], aggi[NPAD:NPAD + NN], node_attr, xi,
            nA1[l], nA2[l], nA3[l], nb1[l], node_W2[l], nb2[l],
            e1a[ln], e1b[ln], i1a[ln], i1b[ln], l == NL - 1)

    pred = _decoder(x, node_mask, sel, dec_W1, db1, dec_W2, db2)
    return pred.reshape(100)
